# baseline (speedup 1.0000x reference)
import jax
import jax.numpy as jnp
from jax import lax
from jax.experimental import pallas as pl
from jax.experimental.pallas import tpu as pltpu
from jax.experimental.pallas import tpu_sc as plsc

B = 16384
EMB = 32
EPS = 1e-5
NC = 2
NS = 16
NW = NC * NS
BPW = B // NW
PASS = 128
GRP = 16


CHUNK = 128
NCH = BPW // CHUNK


def _sc_gmf_body(uidx_hbm, iidx_hbm, ug_hbm, ig_hbm, gmf_out,
                 uidx_v, iidx_v, ug_v, ig_v, sem):
    wid = lax.axis_index("s") * NC + lax.axis_index("c")
    base = wid * BPW
    row0 = wid * NCH
    pltpu.sync_copy(uidx_hbm.at[pl.ds(row0, NCH)], uidx_v)
    pltpu.sync_copy(iidx_hbm.at[pl.ds(row0, NCH)], iidx_v)
    copies = []
    for j in range(NCH):
        dst = pl.ds(j * CHUNK, CHUNK)
        copies.append(pltpu.async_copy(ug_hbm.at[uidx_v.at[j]],
                                       ug_v.at[dst], sem))
        copies.append(pltpu.async_copy(ig_hbm.at[iidx_v.at[j]],
                                       ig_v.at[dst], sem))
    for c in copies:
        c.wait()

    def mul_body(i, carry):
        ug_v[i, pl.ds(0, 16)] = (ug_v[i, pl.ds(0, 16)]
                                 * ig_v[i, pl.ds(0, 16)])
        ug_v[i, pl.ds(16, 16)] = (ug_v[i, pl.ds(16, 16)]
                                  * ig_v[i, pl.ds(16, 16)])
        return carry

    lax.fori_loop(0, BPW, mul_body, 0)
    pltpu.sync_copy(ug_v, gmf_out.at[pl.ds(base, BPW)])


def _sc_gmf(uidx2, iidx2, ug, ig):
    mesh = plsc.VectorSubcoreMesh(core_axis_name="c", subcore_axis_name="s")
    f32 = jnp.float32
    return pl.kernel(
        _sc_gmf_body,
        out_type=pltpu.HBM((B, EMB), f32),
        mesh=mesh,
        scratch_types=[
            pltpu.VMEM((NCH, CHUNK), jnp.int32),
            pltpu.VMEM((NCH, CHUNK), jnp.int32),
            pltpu.VMEM((BPW, EMB), f32),
            pltpu.VMEM((BPW, EMB), f32),
            pltpu.SemaphoreType.DMA,
        ],
        compiler_params=pltpu.CompilerParams(use_tc_tiling_on_sc=False),
    )(uidx2, iidx2, ug, ig)


def _sc_gather_body(uidx_hbm, iidx_hbm, um_hbm, im_hbm,
                    um_out, im_out,
                    uidx_v, iidx_v, um_v, im_v, sem):
    wid = lax.axis_index("s") * NC + lax.axis_index("c")
    base = wid * BPW
    pltpu.sync_copy(uidx_hbm.at[pl.ds(base, BPW)], uidx_v)
    pltpu.sync_copy(iidx_hbm.at[pl.ds(base, BPW)], iidx_v)

    def drain_group():
        pltpu.make_async_copy(um_hbm.at[pl.ds(0, GRP)],
                              um_v.at[pl.ds(0, GRP)], sem).wait()
        pltpu.make_async_copy(im_hbm.at[pl.ds(0, GRP)],
                              im_v.at[pl.ds(0, GRP)], sem).wait()

    for h in range(BPW // PASS):
        off = h * PASS

        def grp(g, carry):
            r0 = g * GRP
            uvec = uidx_v[pl.ds(off + r0, GRP)]
            ivec = iidx_v[pl.ds(off + r0, GRP)]
            for k in range(GRP):
                u = uvec[k]
                it = ivec[k]
                pltpu.async_copy(um_hbm.at[pl.ds(u, 1)],
                                 um_v.at[pl.ds(r0 + k, 1)], sem)
                pltpu.async_copy(im_hbm.at[pl.ds(it, 1)],
                                 im_v.at[pl.ds(r0 + k, 1)], sem)

            @pl.when(g > 0)
            def _():
                drain_group()

            return carry

        lax.fori_loop(0, PASS // GRP, grp, 0)
        drain_group()

        pltpu.sync_copy(um_v, um_out.at[pl.ds(base + off, PASS)])
        pltpu.sync_copy(im_v, im_out.at[pl.ds(base + off, PASS)])


def _sc_gather(uidx, iidx, um, im):
    mesh = plsc.VectorSubcoreMesh(core_axis_name="c", subcore_axis_name="s")
    f32 = jnp.float32
    return pl.kernel(
        _sc_gather_body,
        out_type=[pltpu.HBM((B, EMB), f32)] * 2,
        mesh=mesh,
        scratch_types=[
            pltpu.VMEM((BPW,), jnp.int32),
            pltpu.VMEM((BPW,), jnp.int32),
            pltpu.VMEM((PASS, EMB), f32),
            pltpu.VMEM((PASS, EMB), f32),
            pltpu.SemaphoreType.DMA,
        ],
    )(uidx, iidx, um, im)


def _tc_mlp_body(gmf_ref, um_ref, im_ref, w0u_ref, w0i_ref, b0_ref,
                 w1_ref, b1_ref, w2_ref, b2_ref, pwg_ref, pwm_ref, pb_ref,
                 out_ref):
    f32 = jnp.float32
    x = (jnp.dot(um_ref[...], w0u_ref[...], preferred_element_type=f32)
         + jnp.dot(im_ref[...], w0i_ref[...], preferred_element_type=f32)
         + b0_ref[...])
    x = jnp.maximum(x, 0.0)
    x = jnp.maximum(
        jnp.dot(x, w1_ref[...], preferred_element_type=f32) + b1_ref[...], 0.0)
    x = jnp.maximum(
        jnp.dot(x, w2_ref[...], preferred_element_type=f32) + b2_ref[...], 0.0)
    out = (jnp.sum(gmf_ref[...] * pwg_ref[...], axis=1, keepdims=True)
           + jnp.sum(x * pwm_ref[...], axis=1, keepdims=True)
           + pb_ref[...])
    out_ref[...] = out


def kernel(user_indices, item_indices, user_emb_gmf, item_emb_gmf,
           user_emb_mlp, item_emb_mlp,
           mlp_w0, mlp_b0, bn_g0, bn_b0,
           mlp_w1, mlp_b1, bn_g1, bn_b1,
           mlp_w2, mlp_b2, bn_g2, bn_b2,
           pred_w, pred_b):
    uidx = user_indices.astype(jnp.int32)
    iidx = item_indices.astype(jnp.int32)
    gmf = _sc_gmf(uidx.reshape(B // CHUNK, CHUNK),
                  iidx.reshape(B // CHUNK, CHUNK),
                  user_emb_gmf, item_emb_gmf)
    um, im = _sc_gather(uidx, iidx, user_emb_mlp, item_emb_mlp)

    s0 = bn_g0 / jnp.sqrt(1.0 + EPS)
    s1 = bn_g1 / jnp.sqrt(1.0 + EPS)
    s2 = bn_g2 / jnp.sqrt(1.0 + EPS)
    w0 = mlp_w0.T * s0
    w1 = mlp_w1.T * s1
    w2 = mlp_w2.T * s2
    b0 = (mlp_b0 * s0 + bn_b0).reshape(1, -1)
    b1 = (mlp_b1 * s1 + bn_b1).reshape(1, -1)
    b2 = (mlp_b2 * s2 + bn_b2).reshape(1, -1)
    pwg = pred_w[:, :EMB]
    pwm = pred_w[:, EMB:]
    pb = pred_b.reshape(1, 1)

    out = pl.pallas_call(
        _tc_mlp_body,
        out_shape=jax.ShapeDtypeStruct((B, 1), jnp.float32),
    )(gmf, um, im, w0[:EMB], w0[EMB:], b0, w1, b1, w2, b2, pwg, pwm, pb)
    return out.reshape(-1)


# speedup vs baseline: 1.2466x; 1.2466x over previous
import jax
import jax.numpy as jnp
from jax import lax
from jax.experimental import pallas as pl
from jax.experimental.pallas import tpu as pltpu
from jax.experimental.pallas import tpu_sc as plsc

B = 16384
EMB = 32
EPS = 1e-5
NC = 2
NS = 16
NW = NC * NS
BPW = B // NW
PASS = 128
GRP = 16


def _sc_gather_body(uidx_hbm, iidx_hbm, ug_hbm, ig_hbm, um_hbm, im_hbm,
                    gmf_out, um_out, im_out,
                    uidx_v, iidx_v, ug_v, ig_v, um_v, im_v, sem):
    wid = lax.axis_index("s") * NC + lax.axis_index("c")
    base = wid * BPW
    pltpu.sync_copy(uidx_hbm.at[pl.ds(base, BPW)], uidx_v)
    pltpu.sync_copy(iidx_hbm.at[pl.ds(base, BPW)], iidx_v)

    def drain_group():
        pltpu.make_async_copy(ug_hbm.at[pl.ds(0, GRP)],
                              ug_v.at[pl.ds(0, GRP)], sem).wait()
        pltpu.make_async_copy(ig_hbm.at[pl.ds(0, GRP)],
                              ig_v.at[pl.ds(0, GRP)], sem).wait()
        pltpu.make_async_copy(um_hbm.at[pl.ds(0, GRP)],
                              um_v.at[pl.ds(0, GRP)], sem).wait()
        pltpu.make_async_copy(im_hbm.at[pl.ds(0, GRP)],
                              im_v.at[pl.ds(0, GRP)], sem).wait()

    for h in range(BPW // PASS):
        off = h * PASS

        def grp(g, carry):
            r0 = g * GRP
            uvec = uidx_v[pl.ds(off + r0, GRP)]
            ivec = iidx_v[pl.ds(off + r0, GRP)]
            for k in range(GRP):
                u = uvec[k]
                it = ivec[k]
                pltpu.async_copy(ug_hbm.at[pl.ds(u, 1)],
                                 ug_v.at[pl.ds(r0 + k, 1)], sem)
                pltpu.async_copy(ig_hbm.at[pl.ds(it, 1)],
                                 ig_v.at[pl.ds(r0 + k, 1)], sem)
                pltpu.async_copy(um_hbm.at[pl.ds(u, 1)],
                                 um_v.at[pl.ds(r0 + k, 1)], sem)
                pltpu.async_copy(im_hbm.at[pl.ds(it, 1)],
                                 im_v.at[pl.ds(r0 + k, 1)], sem)

            @pl.when(g > 0)
            def _():
                drain_group()

            return carry

        lax.fori_loop(0, PASS // GRP, grp, 0)
        drain_group()

        def mul_body(i, carry):
            ug_v[i, pl.ds(0, 16)] = (ug_v[i, pl.ds(0, 16)]
                                     * ig_v[i, pl.ds(0, 16)])
            ug_v[i, pl.ds(16, 16)] = (ug_v[i, pl.ds(16, 16)]
                                      * ig_v[i, pl.ds(16, 16)])
            return carry

        lax.fori_loop(0, PASS, mul_body, 0)

        pltpu.sync_copy(ug_v, gmf_out.at[pl.ds(base + off, PASS)])
        pltpu.sync_copy(um_v, um_out.at[pl.ds(base + off, PASS)])
        pltpu.sync_copy(im_v, im_out.at[pl.ds(base + off, PASS)])


def _sc_gather(uidx, iidx, ug, ig, um, im):
    mesh = plsc.VectorSubcoreMesh(core_axis_name="c", subcore_axis_name="s")
    f32 = jnp.float32
    return pl.kernel(
        _sc_gather_body,
        out_type=[pltpu.HBM((B, EMB), f32)] * 3,
        mesh=mesh,
        scratch_types=[
            pltpu.VMEM((BPW,), jnp.int32),
            pltpu.VMEM((BPW,), jnp.int32),
            pltpu.VMEM((PASS, EMB), f32),
            pltpu.VMEM((PASS, EMB), f32),
            pltpu.VMEM((PASS, EMB), f32),
            pltpu.VMEM((PASS, EMB), f32),
            pltpu.SemaphoreType.DMA,
        ],
    )(uidx, iidx, ug, ig, um, im)


def _tc_mlp_body(gmf_ref, um_ref, im_ref, w0u_ref, w0i_ref, b0_ref,
                 w1_ref, b1_ref, w2_ref, b2_ref, pwg_ref, pwm_ref, pb_ref,
                 out_ref):
    f32 = jnp.float32
    x = (jnp.dot(um_ref[...], w0u_ref[...], preferred_element_type=f32)
         + jnp.dot(im_ref[...], w0i_ref[...], preferred_element_type=f32)
         + b0_ref[...])
    x = jnp.maximum(x, 0.0)
    x = jnp.maximum(
        jnp.dot(x, w1_ref[...], preferred_element_type=f32) + b1_ref[...], 0.0)
    x = jnp.maximum(
        jnp.dot(x, w2_ref[...], preferred_element_type=f32) + b2_ref[...], 0.0)
    out = (jnp.sum(gmf_ref[...] * pwg_ref[...], axis=1, keepdims=True)
           + jnp.sum(x * pwm_ref[...], axis=1, keepdims=True)
           + pb_ref[...])
    out_ref[...] = out


def kernel(user_indices, item_indices, user_emb_gmf, item_emb_gmf,
           user_emb_mlp, item_emb_mlp,
           mlp_w0, mlp_b0, bn_g0, bn_b0,
           mlp_w1, mlp_b1, bn_g1, bn_b1,
           mlp_w2, mlp_b2, bn_g2, bn_b2,
           pred_w, pred_b):
    uidx = user_indices.astype(jnp.int32)
    iidx = item_indices.astype(jnp.int32)
    gmf, um, im = _sc_gather(uidx, iidx, user_emb_gmf, item_emb_gmf,
                             user_emb_mlp, item_emb_mlp)

    s0 = bn_g0 / jnp.sqrt(1.0 + EPS)
    s1 = bn_g1 / jnp.sqrt(1.0 + EPS)
    s2 = bn_g2 / jnp.sqrt(1.0 + EPS)
    w0 = mlp_w0.T * s0
    w1 = mlp_w1.T * s1
    w2 = mlp_w2.T * s2
    b0 = (mlp_b0 * s0 + bn_b0).reshape(1, -1)
    b1 = (mlp_b1 * s1 + bn_b1).reshape(1, -1)
    b2 = (mlp_b2 * s2 + bn_b2).reshape(1, -1)
    pwg = pred_w[:, :EMB]
    pwm = pred_w[:, EMB:]
    pb = pred_b.reshape(1, 1)

    out = pl.pallas_call(
        _tc_mlp_body,
        out_shape=jax.ShapeDtypeStruct((B, 1), jnp.float32),
    )(gmf, um, im, w0[:EMB], w0[EMB:], b0, w1, b1, w2, b2, pwg, pwm, pb)
    return out.reshape(-1)
